# baseline (speedup 1.0000x reference)
import jax
import jax.numpy as jnp
from jax.experimental import pallas as pl
from jax.experimental.pallas import tpu as pltpu

_EPS = 1e-5
_BC = 32


def _fused_kernel(x_ref, seg_ref,
                  tw0, tb0, tw1, tb1, tw2, tb2, tw3, tb3,
                  gw0, gb0, gw1, gb1, gw2, gb2, gw3, gb3,
                  bw0, bb0, bw1, bb1, bw2, bb2, bw3, bb3,
                  o_ref):
    def lin(a, w, bias):
        return jnp.dot(a, w[...], preferred_element_type=jnp.float32) + bias[...]

    h = jax.nn.relu(lin(seg_ref[0], tw0, tb0))
    h = jax.nn.relu(lin(h, tw1, tb1))
    h = jax.nn.relu(lin(h, tw2, tb2))
    h = jax.nn.relu(lin(h, tw3, tb3))

    g = jax.nn.relu(lin(h, gw0, gb0))
    g = jax.nn.relu(lin(g, gw1, gb1))
    g = jax.nn.relu(lin(g, gw2, gb2))
    g = lin(g, gw3, gb3)

    b = jax.nn.relu(lin(h, bw0, bb0))
    b = jax.nn.relu(lin(b, bw1, bb1))
    b = jax.nn.relu(lin(b, bw2, bb2))
    b = lin(b, bw3, bb3)

    scale = 1.0 + jnp.transpose(g)
    shift = jnp.transpose(b)

    xb = x_ref[0]
    mean = jnp.mean(xb, axis=(1, 2), keepdims=True)
    d = xb - mean
    var = jnp.mean(d * d, axis=(1, 2), keepdims=True)
    r = jax.lax.rsqrt(var + _EPS)
    o_ref[0] = (d * r) * scale[None] + shift[None]


@jax.jit
def kernel(x, segmap, tw0, tb0, tw1, tb1, tw2, tb2, tw3, tb3,
           gw0, gb0, gw1, gb1, gw2, gb2, gw3, gb3,
           bw0, bb0, bw1, bb1, bw2, bb2, bw3, bb3):
    n, c, h, w = x.shape
    seg3 = segmap.reshape(n, 1, segmap.shape[-1])
    biases2d = [t.reshape(1, -1) for t in
                (tb0, tb1, tb2, tb3, gb0, gb1, gb2, gb3, bb0, bb1, bb2, bb3)]
    weights = (tw0, tw1, tw2, tw3, gw0, gw1, gw2, gw3, bw0, bw1, bw2, bw3)

    wspecs = []
    for wt in weights:
        wspecs.append(pl.BlockSpec(wt.shape, lambda i, j: (0, 0)))
        wspecs.append(pl.BlockSpec((1, h), lambda i, j: (0, 0)))

    out = pl.pallas_call(
        _fused_kernel,
        out_shape=jax.ShapeDtypeStruct((n, c, h, w), jnp.float32),
        grid=(n, c // _BC),
        in_specs=[
            pl.BlockSpec((1, _BC, h, w), lambda i, j: (i, j, 0, 0)),
            pl.BlockSpec((1, 1, seg3.shape[-1]), lambda i, j: (i, 0, 0)),
        ] + wspecs,
        out_specs=pl.BlockSpec((1, _BC, h, w), lambda i, j: (i, j, 0, 0)),
        compiler_params=pltpu.CompilerParams(
            dimension_semantics=("parallel", "parallel"),
            vmem_limit_bytes=56 * 1024 * 1024,
        ),
        name="spade_instnorm_fused",
    )(x, seg3,
      tw0, biases2d[0], tw1, biases2d[1], tw2, biases2d[2], tw3, biases2d[3],
      gw0, biases2d[4], gw1, biases2d[5], gw2, biases2d[6], gw3, biases2d[7],
      bw0, biases2d[8], bw1, biases2d[9], bw2, biases2d[10], bw3, biases2d[11])
    return out


# speedup vs baseline: 1.0066x; 1.0066x over previous
import jax
import jax.numpy as jnp
from jax.experimental import pallas as pl
from jax.experimental.pallas import tpu as pltpu

_EPS = 1e-5
_BC = 32


def _mlp_kernel(seg_ref, tw0, tb0, tw1, tb1, tw2, tb2, tw3, tb3,
                gw0, gb0, gw1, gb1, gw2, gb2, gw3, gb3,
                bw0, bb0, bw1, bb1, bw2, bb2, bw3, bb3,
                g_ref, b_ref):
    def lin(a, w, bias):
        return jnp.dot(a, w[...], preferred_element_type=jnp.float32) + bias[...]

    h = jax.nn.relu(lin(seg_ref[...], tw0, tb0))
    h = jax.nn.relu(lin(h, tw1, tb1))
    h = jax.nn.relu(lin(h, tw2, tb2))
    h = jax.nn.relu(lin(h, tw3, tb3))

    g = jax.nn.relu(lin(h, gw0, gb0))
    g = jax.nn.relu(lin(g, gw1, gb1))
    g = jax.nn.relu(lin(g, gw2, gb2))
    g = lin(g, gw3, gb3)

    b = jax.nn.relu(lin(h, bw0, bb0))
    b = jax.nn.relu(lin(b, bw1, bb1))
    b = jax.nn.relu(lin(b, bw2, bb2))
    b = lin(b, bw3, bb3)

    g_ref[...] = g[:, None, :]
    b_ref[...] = b[:, None, :]


def _norm_kernel(x_ref, g_ref, b_ref, o_ref):
    xb = x_ref[0]
    mean = jnp.mean(xb, axis=(1, 2), keepdims=True)
    d = xb - mean
    var = jnp.mean(d * d, axis=(1, 2), keepdims=True)
    r = jax.lax.rsqrt(var + _EPS)

    scale = 1.0 + jnp.transpose(g_ref[0])
    shift = jnp.transpose(b_ref[0])
    o_ref[0] = (d * r) * scale[None] + shift[None]


@jax.jit
def kernel(x, segmap, tw0, tb0, tw1, tb1, tw2, tb2, tw3, tb3,
           gw0, gb0, gw1, gb1, gw2, gb2, gw3, gb3,
           bw0, bb0, bw1, bb1, bw2, bb2, bw3, bb3):
    n, c, h, w = x.shape
    biases2d = [t.reshape(1, -1) for t in
                (tb0, tb1, tb2, tb3, gb0, gb1, gb2, gb3, bb0, bb1, bb2, bb3)]

    g3, b3 = pl.pallas_call(
        _mlp_kernel,
        out_shape=(jax.ShapeDtypeStruct((n, 1, h), jnp.float32),
                   jax.ShapeDtypeStruct((n, 1, h), jnp.float32)),
        name="spade_mlp",
    )(segmap,
      tw0, biases2d[0], tw1, biases2d[1], tw2, biases2d[2], tw3, biases2d[3],
      gw0, biases2d[4], gw1, biases2d[5], gw2, biases2d[6], gw3, biases2d[7],
      bw0, biases2d[8], bw1, biases2d[9], bw2, biases2d[10], bw3, biases2d[11])

    out = pl.pallas_call(
        _norm_kernel,
        out_shape=jax.ShapeDtypeStruct((n, c, h, w), jnp.float32),
        grid=(n, c // _BC),
        in_specs=[
            pl.BlockSpec((1, _BC, h, w), lambda i, j: (i, j, 0, 0)),
            pl.BlockSpec((1, 1, h), lambda i, j: (i, 0, 0)),
            pl.BlockSpec((1, 1, h), lambda i, j: (i, 0, 0)),
        ],
        out_specs=pl.BlockSpec((1, _BC, h, w), lambda i, j: (i, j, 0, 0)),
        compiler_params=pltpu.CompilerParams(
            dimension_semantics=("parallel", "parallel"),
            vmem_limit_bytes=56 * 1024 * 1024,
        ),
        name="instnorm_affine",
    )(x, g3, b3)
    return out


# speedup vs baseline: 1.0071x; 1.0005x over previous
import jax
import jax.numpy as jnp
from jax.experimental import pallas as pl
from jax.experimental.pallas import tpu as pltpu

_EPS = 1e-5
_BC = 32


def _mlp_kernel(seg_ref, tw0, tb0, tw1, tb1, tw2, tb2, tw3, tb3,
                gw0, gb0, gw1, gb1, gw2, gb2, gw3, gb3,
                bw0, bb0, bw1, bb1, bw2, bb2, bw3, bb3,
                g_ref, b_ref):
    def lin(a, w, bias):
        return jnp.dot(a, w[...], preferred_element_type=jnp.float32) + bias[...]

    h = jax.nn.relu(lin(seg_ref[...], tw0, tb0))
    h = jax.nn.relu(lin(h, tw1, tb1))
    h = jax.nn.relu(lin(h, tw2, tb2))
    h = jax.nn.relu(lin(h, tw3, tb3))

    g = jax.nn.relu(lin(h, gw0, gb0))
    g = jax.nn.relu(lin(g, gw1, gb1))
    g = jax.nn.relu(lin(g, gw2, gb2))
    g = lin(g, gw3, gb3)

    b = jax.nn.relu(lin(h, bw0, bb0))
    b = jax.nn.relu(lin(b, bw1, bb1))
    b = jax.nn.relu(lin(b, bw2, bb2))
    b = lin(b, bw3, bb3)

    g_ref[...] = g[:, None, :]
    b_ref[...] = b[:, None, :]


def _norm_kernel(x_ref, g_ref, b_ref, o_ref):
    xb = x_ref[0]
    mean = jnp.mean(xb, axis=(1, 2), keepdims=True)
    d = xb - mean
    var = jnp.mean(d * d, axis=(1, 2), keepdims=True)
    r = jax.lax.rsqrt(var + _EPS)

    scale = 1.0 + jnp.transpose(g_ref[0])
    shift = jnp.transpose(b_ref[0])
    o_ref[0] = (d * r) * scale[None] + shift[None]


@jax.jit
def kernel(x, segmap, tw0, tb0, tw1, tb1, tw2, tb2, tw3, tb3,
           gw0, gb0, gw1, gb1, gw2, gb2, gw3, gb3,
           bw0, bb0, bw1, bb1, bw2, bb2, bw3, bb3):
    n, c, h, w = x.shape
    biases2d = [t.reshape(1, -1) for t in
                (tb0, tb1, tb2, tb3, gb0, gb1, gb2, gb3, bb0, bb1, bb2, bb3)]

    g3, b3 = pl.pallas_call(
        _mlp_kernel,
        out_shape=(jax.ShapeDtypeStruct((n, 1, h), jnp.float32),
                   jax.ShapeDtypeStruct((n, 1, h), jnp.float32)),
        name="spade_mlp",
    )(segmap,
      tw0, biases2d[0], tw1, biases2d[1], tw2, biases2d[2], tw3, biases2d[3],
      gw0, biases2d[4], gw1, biases2d[5], gw2, biases2d[6], gw3, biases2d[7],
      bw0, biases2d[8], bw1, biases2d[9], bw2, biases2d[10], bw3, biases2d[11])

    out = pl.pallas_call(
        _norm_kernel,
        out_shape=jax.ShapeDtypeStruct((n, c, h, w), jnp.float32),
        grid=(n * (c // _BC),),
        in_specs=[
            pl.BlockSpec((1, _BC, h, w), lambda k: (k // (c // _BC), k % (c // _BC), 0, 0)),
            pl.BlockSpec((1, 1, h), lambda k: (k // (c // _BC), 0, 0)),
            pl.BlockSpec((1, 1, h), lambda k: (k // (c // _BC), 0, 0)),
        ],
        out_specs=pl.BlockSpec((1, _BC, h, w), lambda k: (k // (c // _BC), k % (c // _BC), 0, 0)),
        compiler_params=pltpu.CompilerParams(
            dimension_semantics=("parallel",),
            vmem_limit_bytes=56 * 1024 * 1024,
        ),
        name="instnorm_affine",
    )(x, g3, b3)
    return out
